# Initial kernel scaffold; baseline (speedup 1.0000x reference)
import jax
import jax.numpy as jnp
from jax.experimental import pallas as pl


def kernel(x, n):
    raise NotImplementedError("write your pallas kernel here")


# speedup vs baseline: 3.8289x; 3.8289x over previous
import functools

import jax
import jax.numpy as jnp
import numpy as np
from jax.experimental import pallas as pl
from jax.experimental.pallas import tpu as pltpu

_SIGN = np.int32(-2147483648)
_MAG = np.int32(2147483647)


def _ukey(xf):
    xi = jax.lax.bitcast_convert_type(xf, jnp.int32)
    key = jnp.where(xi >= 0, xi, jnp.bitwise_xor(xi, _MAG))
    return jnp.bitwise_xor(key, _SIGN)


def _select_kernel(x_ref, n_ref, thr_ref, hist_ref, sm_ref):
    l = pl.program_id(0)
    i = pl.program_id(1)
    nblocks = pl.num_programs(1)
    c = x_ref.shape[2]

    @pl.when(jnp.logical_and(l == 0, i == 0))
    def _init():
        sm_ref[0] = jnp.int32(0)
        sm_ref[1] = n_ref[0]

    @pl.when(i == 0)
    def _zero():
        hist_ref[...] = jnp.zeros_like(hist_ref)

    uk = _ukey(x_ref[0])
    shift = 24 - 8 * l
    pfx = sm_ref[0]
    lvl_mask = jax.lax.shift_left(jnp.int32(1), 8 * l) - 1
    vshift = jnp.minimum(shift + 8, 31)
    valid = jnp.bitwise_and(
        jax.lax.shift_right_arithmetic(uk, vshift), lvl_mask) == pfx
    d = jnp.bitwise_and(jax.lax.shift_right_arithmetic(uk, shift), 255)
    hi4 = jax.lax.shift_right_logical(d, 4)
    lo4 = jnp.bitwise_and(d, 15)

    biota = jax.lax.broadcasted_iota(jnp.int32, (16, c), 0)
    a = jnp.logical_and(hi4 == biota, valid).astype(jnp.float32)
    b = (lo4 == biota).astype(jnp.float32)
    contrib = jax.lax.dot_general(
        a, b, (((1,), (1,)), ((), ())), preferred_element_type=jnp.float32)
    hist_ref[...] += contrib

    @pl.when(i == nblocks - 1)
    def _select():
        h = hist_ref[...]
        ii = jax.lax.broadcasted_iota(jnp.int32, (16, 16), 0)
        jj = jax.lax.broadcasted_iota(jnp.int32, (16, 16), 1)
        rowsum = jnp.sum(h, axis=1, keepdims=True)
        tstrict = (ii > jj).astype(jnp.float32)
        prev = jax.lax.dot_general(
            tstrict, rowsum, (((1,), (0,)), ((), ())),
            preferred_element_type=jnp.float32,
            precision=jax.lax.Precision.HIGHEST)
        uincl = (ii <= jj).astype(jnp.float32)
        rowcum = jax.lax.dot_general(
            h, uincl, (((1,), (0,)), ((), ())),
            preferred_element_type=jnp.float32,
            precision=jax.lax.Precision.HIGHEST)
        cum = rowcum + prev
        r = sm_ref[1]
        rf = r.astype(jnp.float32)
        flat = ii * 16 + jj
        bsel = jnp.min(jnp.where(cum >= rf, flat, 256))
        hb = jnp.sum(jnp.where(flat == bsel, h, 0.0))
        cumb = jnp.sum(jnp.where(flat == bsel, cum, 0.0))
        excl = cumb - hb
        sm_ref[1] = r - excl.astype(jnp.int32)
        newpfx = jnp.bitwise_or(jax.lax.shift_left(pfx, 8), bsel)
        sm_ref[0] = newpfx

        @pl.when(l == 3)
        def _emit():
            ukt = jnp.full((1, 1), newpfx, jnp.int32)
            keyt = jnp.bitwise_xor(ukt, _SIGN)
            it = jnp.where(keyt >= 0, keyt, jnp.bitwise_xor(keyt, _MAG))
            thr_ref[...] = jax.lax.bitcast_convert_type(it, jnp.float32)


def _mask_kernel(x_ref, thr_ref, o_ref):
    o_ref[...] = (x_ref[...] >= thr_ref[0]).astype(jnp.float32)


@functools.partial(jax.jit, static_argnames=())
def kernel(x, n):
    total = x.shape[0]
    c = 8192
    nb = total // c
    x3 = x.reshape(nb, 1, c)
    narr = jnp.asarray(n, jnp.int32).reshape(1)

    thr = pl.pallas_call(
        _select_kernel,
        grid=(4, nb),
        in_specs=[
            pl.BlockSpec((1, 1, c), lambda l, i: (i, 0, 0)),
            pl.BlockSpec(memory_space=pltpu.SMEM),
        ],
        out_specs=pl.BlockSpec((1, 1), lambda l, i: (0, 0)),
        out_shape=jax.ShapeDtypeStruct((1, 1), jnp.float32),
        scratch_shapes=[
            pltpu.VMEM((16, 16), jnp.float32),
            pltpu.SMEM((2,), jnp.int32),
        ],
    )(x3, narr)

    rows = 512
    x2 = x.reshape(rows, total // rows)
    blk = 8
    out = pl.pallas_call(
        _mask_kernel,
        grid=(rows // blk,),
        in_specs=[
            pl.BlockSpec((blk, total // rows), lambda i: (i, 0)),
            pl.BlockSpec(memory_space=pltpu.SMEM),
        ],
        out_specs=pl.BlockSpec((blk, total // rows), lambda i: (i, 0)),
        out_shape=jax.ShapeDtypeStruct((rows, total // rows), jnp.float32),
    )(x2, thr.reshape(1))
    return out.reshape(total)


# speedup vs baseline: 9.8295x; 2.5672x over previous
import functools

import jax
import jax.numpy as jnp
import numpy as np
from jax import lax
from jax.experimental import pallas as pl
from jax.experimental.pallas import tpu as pltpu
from jax.experimental.pallas import tpu_sc as plsc

_SIGN = np.int32(-2147483648)
_MAG = np.int32(2147483647)

_N = 4194304
_NC = 2
_NS = 16
_NW = _NC * _NS
_PER_TILE = _N // _NW
_CHUNK = 16384
_NCHUNK = _PER_TILE // _CHUNK
_BINS = 2048
_NVR = _BINS // 16


def _ukey16(v):
    xi = lax.bitcast_convert_type(v, jnp.int32)
    key = jnp.where(xi >= 0, xi, jnp.bitwise_xor(xi, _MAG))
    return jnp.bitwise_xor(key, _SIGN)


def _lane_scalar(vec, lane, k):
    return jnp.sum(jnp.where(lane == k, vec, 0))


def _make_sc_level(level, bits, shift, prev_bits):
    mesh = plsc.VectorSubcoreMesh(core_axis_name="c", subcore_axis_name="s")
    dmask = np.int32((1 << bits) - 1)
    if level > 0:
        vshift = shift + bits
        vmask = np.int32((1 << (32 - vshift)) - 1) if vshift < 32 else np.int32(0)

    def body(x_hbm, *rest):
        if level > 0:
            (hprev_hbm, state_hbm, hout_hbm, stout_hbm,
             buf, hist, hsum, hpbuf, stv) = rest
        else:
            (hout_hbm, buf, hist, hsum, hpbuf, stv) = rest
        c = lax.axis_index("c")
        s = lax.axis_index("s")
        wid = c * _NS + s
        lane = lax.broadcasted_iota(jnp.int32, (16,), 0)
        zeros16 = jnp.zeros((16,), jnp.int32)
        ones16 = jnp.full((16,), 1, jnp.int32)

        if level > 0:
            pltpu.sync_copy(state_hbm, stv)
            sv = stv[...]
            pfx_prev = _lane_scalar(sv, lane, 0)
            r = _lane_scalar(sv, lane, 1)

            def mzero(vb, _):
                hsum[pl.ds(vb * 16, 16)] = zeros16
                return 0
            lax.fori_loop(0, _NVR, mzero, 0)
            for g in range(4):
                pltpu.sync_copy(hprev_hbm.at[pl.ds(g * 8, 8)], hpbuf)

                def macc(vb, _):
                    acc = hsum[pl.ds(vb * 16, 16)]
                    for t in range(8):
                        acc = acc + hpbuf[t, pl.ds(vb * 16, 16)]
                    hsum[pl.ds(vb * 16, 16)] = acc
                    return 0
                lax.fori_loop(0, _NVR, macc, 0)

            def sel_body(vb, carry):
                cum, found, bsel, excl = carry
                v = hsum[pl.ds(vb * 16, 16)]
                cs = plsc.cumsum(v) + cum
                m = cs >= r
                pc = plsc.all_reduce_population_count(m)
                ffs = plsc.all_reduce_ffs(m)
                selnow = jnp.logical_and(found == 0, pc > 0)
                bsel = jnp.where(selnow, vb * 16 + ffs, bsel)
                exl = jnp.sum(jnp.where(lane == ffs, cs - v, 0))
                excl = jnp.where(selnow, exl, excl)
                found = jnp.where(pc > 0, ones16, found)
                tot = jnp.sum(jnp.where(lane == 15, cs, 0))
                return (zeros16 + tot, found, bsel, excl)

            cum0 = (zeros16, zeros16, zeros16, zeros16)
            _, _, bsel, excl = lax.fori_loop(0, _NVR, sel_body, cum0)
            pfx = jnp.bitwise_or(lax.shift_left(pfx_prev, prev_bits), bsel)
            rnew = r - excl

            @pl.when(wid == 0)
            def _wr_state():
                stv[...] = jnp.where(lane == 0, pfx,
                                     jnp.where(lane == 1, rnew, 0))
                pltpu.sync_copy(stv, stout_hbm)

        def zbody(i, _):
            for u in range(8):
                hist[pl.ds((i * 8 + u) * 16, 16)] = zeros16
            return 0
        lax.fori_loop(0, 16 * _NVR // 8, zbody, 0)

        lane_base = lane * _BINS

        def cbody(i, _):
            for u in range(4):
                v = buf[pl.ds((i * 4 + u) * 16, 16)]
                uk = _ukey16(v)
                d = jnp.bitwise_and(
                    lax.shift_right_arithmetic(uk, shift), dmask)
                idx = lane_base + d
                cur = plsc.load_gather(hist, [idx])
                if level > 0:
                    hi = jnp.bitwise_and(
                        lax.shift_right_arithmetic(uk, vshift), vmask)
                    inc = jnp.where(hi == pfx, ones16, zeros16)
                else:
                    inc = ones16
                plsc.store_scatter(hist, [idx], cur + inc)
            return 0

        for j in range(_NCHUNK):
            base = wid * _PER_TILE + j * _CHUNK
            pltpu.sync_copy(x_hbm.at[pl.ds(base, _CHUNK)], buf)
            lax.fori_loop(0, _CHUNK // 64, cbody, 0)

        def rbody(vb, _):
            acc = zeros16
            for row in range(16):
                acc = acc + hist[pl.ds(row * _BINS + vb * 16, 16)]
            hsum[pl.ds(vb * 16, 16)] = acc
            return 0
        lax.fori_loop(0, _NVR, rbody, 0)
        pltpu.sync_copy(hsum, hout_hbm.at[wid])

    out_type = [jax.ShapeDtypeStruct((_NW, _BINS), jnp.int32)]
    if level > 0:
        out_type.append(jax.ShapeDtypeStruct((16,), jnp.int32))
    scratch = [
        pltpu.VMEM((_CHUNK,), jnp.float32),
        pltpu.VMEM((16 * _BINS,), jnp.int32),
        pltpu.VMEM((_BINS,), jnp.int32),
        pltpu.VMEM((8, _BINS), jnp.int32),
        pltpu.VMEM((16,), jnp.int32),
    ]
    return pl.kernel(
        body, out_type=out_type, mesh=mesh, scratch_types=scratch,
        compiler_params=pltpu.CompilerParams(needs_layout_passes=False))


_sc_l0 = _make_sc_level(0, 11, 21, 0)
_sc_l1 = _make_sc_level(1, 11, 10, 11)
_sc_l2 = _make_sc_level(2, 10, 0, 11)


def _mask_kernel(x_ref, h_ref, st_ref, o_ref, thr_ref):
    i = pl.program_id(0)

    @pl.when(i == 0)
    def _select():
        h = jnp.sum(h_ref[...].astype(jnp.float32), axis=0)
        ii = lax.broadcasted_iota(jnp.int32, (64, 64), 0)
        jj = lax.broadcasted_iota(jnp.int32, (64, 64), 1)
        rowsum = jnp.sum(h, axis=1, keepdims=True)
        tstrict = (ii > jj).astype(jnp.float32)
        prev = lax.dot_general(
            tstrict, rowsum, (((1,), (0,)), ((), ())),
            preferred_element_type=jnp.float32,
            precision=lax.Precision.HIGHEST)
        ii2 = lax.broadcasted_iota(jnp.int32, (32, 32), 0)
        jj2 = lax.broadcasted_iota(jnp.int32, (32, 32), 1)
        uincl = (ii2 <= jj2).astype(jnp.float32)
        rowcum = lax.dot_general(
            h, uincl, (((1,), (0,)), ((), ())),
            preferred_element_type=jnp.float32,
            precision=lax.Precision.HIGHEST)
        cum = rowcum + prev
        r = st_ref[1]
        iif = lax.broadcasted_iota(jnp.int32, (64, 32), 0)
        jjf = lax.broadcasted_iota(jnp.int32, (64, 32), 1)
        flat = iif * 32 + jjf
        bsel = jnp.min(jnp.where(cum >= r.astype(jnp.float32), flat, 2048))
        ukt = jnp.bitwise_or(lax.shift_left(st_ref[0], 10), bsel)
        keyt = jnp.bitwise_xor(jnp.full((1, 1), ukt, jnp.int32), _SIGN)
        it = jnp.where(keyt >= 0, keyt, jnp.bitwise_xor(keyt, _MAG))
        thr_ref[0] = lax.bitcast_convert_type(it, jnp.float32)[0, 0]

    o_ref[...] = (x_ref[...] >= thr_ref[0]).astype(jnp.float32)


@jax.jit
def kernel(x, n):
    total = x.shape[0]
    state0 = jnp.zeros((16,), jnp.int32).at[1].set(n)

    (h0,) = _sc_l0(x)
    h1, state1 = _sc_l1(x, h0, state0)
    h2, state2 = _sc_l2(x, h1, state1)

    h2r = h2.reshape(_NW, 64, 32)
    rows = 512
    cols = total // rows
    x2 = x.reshape(rows, cols)
    blk = 8
    out = pl.pallas_call(
        _mask_kernel,
        grid=(rows // blk,),
        in_specs=[
            pl.BlockSpec((blk, cols), lambda i: (i, 0)),
            pl.BlockSpec((_NW, 64, 32), lambda i: (0, 0, 0)),
            pl.BlockSpec(memory_space=pltpu.SMEM),
        ],
        out_specs=pl.BlockSpec((blk, cols), lambda i: (i, 0)),
        out_shape=jax.ShapeDtypeStruct((rows, cols), jnp.float32),
        scratch_shapes=[pltpu.SMEM((1,), jnp.float32)],
    )(x2, h2r, state2)
    return out.reshape(total)
